# baseline (speedup 1.0000x reference)
import jax
import jax.numpy as jnp
from jax.experimental import pallas as pl

_N = 512
_P = 64
_E = 128


def _chunk16(pT_c, xsT_c):
    r = pT_c[:, None, :] - xsT_c[:, :, None]
    u = r[0:8] + r[8:16]
    u = u[0:4] + u[4:8]
    u = u[0:2] + u[2:4]
    return u[0] + u[1]


def _tree_sum_p(pT, xsT):
    s0 = _chunk16(pT[0:16], xsT[0:16])
    s1 = _chunk16(pT[16:32], xsT[16:32])
    s2 = _chunk16(pT[32:48], xsT[32:48])
    s3 = _chunk16(pT[48:64], xsT[48:64])
    return ((s0 + s1) + s2) + s3


def _body(x_ref, permT_ref, ehi_ref, emid_ref, elo_ref, out_ref):
    xb = x_ref[0]
    h, wb, p = xb.shape
    t_tok = h * wb
    xmin = jnp.min(xb, axis=0, keepdims=True)
    xmax = jnp.max(xb, axis=0, keepdims=True)
    den = (xmax - xmin) + jnp.float32(1e-8)
    xs = (xb - xmin) / den
    xs2 = xs.reshape(t_tok, p)
    xsT = xs2.T
    pT = permT_ref[...]
    t = _tree_sum_p(pT, xsT)
    at = jnp.abs(t)
    m = jnp.min(at, axis=1, keepdims=True)
    ii = jax.lax.broadcasted_iota(jnp.int32, at.shape, 1)
    idx = jnp.min(jnp.where(at == m, ii, _N), axis=1)
    oh = (jax.lax.broadcasted_iota(jnp.int32, (t_tok, _N), 1)
          == idx[:, None]).astype(jnp.bfloat16)
    dn = (((1,), (0,)), ((), ()))
    ob = (jax.lax.dot_general(oh, ehi_ref[...], dn,
                              preferred_element_type=jnp.float32)
          + jax.lax.dot_general(oh, emid_ref[...], dn,
                                preferred_element_type=jnp.float32)) \
         + jax.lax.dot_general(oh, elo_ref[...], dn,
                               preferred_element_type=jnp.float32)
    out_ref[...] = ob.reshape(1, h, wb, _E)


def kernel(x, perm, emb):
    b, h, w, p = x.shape
    permT = perm.reshape(_N, _P).T
    ehi = emb.astype(jnp.bfloat16)
    rem = emb - ehi.astype(jnp.float32)
    emid = rem.astype(jnp.bfloat16)
    elo = (rem - emid.astype(jnp.float32)).astype(jnp.bfloat16)
    grid = (b,)
    return pl.pallas_call(
        _body,
        grid=grid,
        in_specs=[
            pl.BlockSpec((1, h, w, p), lambda i: (i, 0, 0, 0)),
            pl.BlockSpec((_P, _N), lambda i: (0, 0)),
            pl.BlockSpec((_N, _E), lambda i: (0, 0)),
            pl.BlockSpec((_N, _E), lambda i: (0, 0)),
            pl.BlockSpec((_N, _E), lambda i: (0, 0)),
        ],
        out_specs=pl.BlockSpec((1, h, w, _E), lambda i: (i, 0, 0, 0)),
        out_shape=jax.ShapeDtypeStruct((b, h, w, _E), jnp.float32),
    )(x, permT, ehi, emid, elo)


# speedup vs baseline: 1.0098x; 1.0098x over previous
import jax
import jax.numpy as jnp
from jax.experimental import pallas as pl

_N = 512
_P = 64
_E = 128


def _chunk16(pT_c, xsT_c):
    r = pT_c[:, None, :] - xsT_c[:, :, None]
    u = r[0:8] + r[8:16]
    u = u[0:4] + u[4:8]
    u = u[0:2] + u[2:4]
    return u[0] + u[1]


def _tree_sum_p(pT, xsT):
    s0 = _chunk16(pT[0:16], xsT[0:16])
    s1 = _chunk16(pT[16:32], xsT[16:32])
    s2 = _chunk16(pT[32:48], xsT[32:48])
    s3 = _chunk16(pT[48:64], xsT[48:64])
    return ((s0 + s1) + s2) + s3


def _body(x_ref, permT_ref, emb_ref, out_ref):
    xb = x_ref[0]
    h, wb, p = xb.shape
    t_tok = h * wb
    xmin = jnp.min(xb, axis=0, keepdims=True)
    xmax = jnp.max(xb, axis=0, keepdims=True)
    den = (xmax - xmin) + jnp.float32(1e-8)
    xs = (xb - xmin) / den
    xs2 = xs.reshape(t_tok, p)
    xsT = xs2.T
    pT = permT_ref[...]
    t = _tree_sum_p(pT, xsT)
    at = jnp.abs(t)
    m = jnp.min(at, axis=1, keepdims=True)
    ii = jax.lax.broadcasted_iota(jnp.int32, at.shape, 1)
    idx = jnp.min(jnp.where(at == m, ii, _N), axis=1)
    oh = (jax.lax.broadcasted_iota(jnp.int32, (t_tok, _N), 1)
          == idx[:, None]).astype(jnp.float32)
    ob = jax.lax.dot_general(oh, emb_ref[...],
                             (((1,), (0,)), ((), ())),
                             preferred_element_type=jnp.float32,
                             precision=jax.lax.Precision.HIGHEST)
    out_ref[...] = ob.reshape(1, h, wb, _E)


def kernel(x, perm, emb):
    b, h, w, p = x.shape
    permT = perm.reshape(_N, _P).T
    grid = (b,)
    return pl.pallas_call(
        _body,
        grid=grid,
        in_specs=[
            pl.BlockSpec((1, h, w, p), lambda i: (i, 0, 0, 0)),
            pl.BlockSpec((_P, _N), lambda i: (0, 0)),
            pl.BlockSpec((_N, _E), lambda i: (0, 0)),
        ],
        out_specs=pl.BlockSpec((1, h, w, _E), lambda i: (i, 0, 0, 0)),
        out_shape=jax.ShapeDtypeStruct((b, h, w, _E), jnp.float32),
    )(x, permT, emb)


# speedup vs baseline: 1.0255x; 1.0156x over previous
import jax
import jax.numpy as jnp
from jax.experimental import pallas as pl

_N = 512
_P = 64
_E = 128


def _chunk16(pT_c, xsT_c):
    r = pT_c[:, None, :] - xsT_c[:, :, None]
    u = r[0:8] + r[8:16]
    u = u[0:4] + u[4:8]
    u = u[0:2] + u[2:4]
    return u[0] + u[1]


def _tree_sum_p(pT, xsT):
    s0 = _chunk16(pT[0:16], xsT[0:16])
    s1 = _chunk16(pT[16:32], xsT[16:32])
    s2 = _chunk16(pT[32:48], xsT[32:48])
    s3 = _chunk16(pT[48:64], xsT[48:64])
    return ((s0 + s1) + s2) + s3


def _body(x_ref, permT_ref, emb_ref, out_ref):
    xb = x_ref[...]
    bb, h, wb, p = xb.shape
    t_tok = bb * h * wb
    xmin = jnp.min(xb, axis=1, keepdims=True)
    xmax = jnp.max(xb, axis=1, keepdims=True)
    den = (xmax - xmin) + jnp.float32(1e-8)
    xs = (xb - xmin) / den
    xs2 = xs.reshape(t_tok, p)
    xsT = xs2.T
    pT = permT_ref[...]
    t = _tree_sum_p(pT, xsT)
    at = jnp.abs(t)
    m = jnp.min(at, axis=1, keepdims=True)
    ii = jax.lax.broadcasted_iota(jnp.int32, at.shape, 1)
    idx = jnp.min(jnp.where(at == m, ii, _N), axis=1)
    oh = (jax.lax.broadcasted_iota(jnp.int32, (t_tok, _N), 1)
          == idx[:, None]).astype(jnp.float32)
    ob = jax.lax.dot_general(oh, emb_ref[...],
                             (((1,), (0,)), ((), ())),
                             preferred_element_type=jnp.float32,
                             precision=jax.lax.Precision.HIGHEST)
    out_ref[...] = ob.reshape(bb, h, wb, _E)


def kernel(x, perm, emb):
    b, h, w, p = x.shape
    permT = perm.reshape(_N, _P).T
    return pl.pallas_call(
        _body,
        out_shape=jax.ShapeDtypeStruct((b, h, w, _E), jnp.float32),
    )(x, permT, emb)
